# baseline (speedup 1.0000x reference)
import functools

import jax
import jax.numpy as jnp
from jax import lax
from jax.experimental import pallas as pl
from jax.experimental.pallas import tpu as pltpu
from jax.experimental.pallas import tpu_sc as plsc

NCORES = 2
NSUB = 16
NW = NCORES * NSUB
CHUNK = 64


def _sc_mesh():
  return plsc.VectorSubcoreMesh(
      core_axis_name="c", subcore_axis_name="s",
      num_cores=NCORES, num_subcores=NSUB)


def _deg_partials(colf, zeros_row, nacc, chunks):
  epw = chunks * CHUNK
  groups = epw // 16

  @functools.partial(
      pl.kernel, mesh=_sc_mesh(),
      out_type=jax.ShapeDtypeStruct((NW, nacc), jnp.float32),
      compiler_params=pltpu.CompilerParams(needs_layout_passes=False),
      scratch_types=[
          pltpu.VMEM((epw,), jnp.int32),
          pltpu.VMEM((nacc,), jnp.float32),
      ])
  def k(col_hbm, z_hbm, out_hbm, col_v, hist):
    c = lax.axis_index("c")
    s = lax.axis_index("s")
    wid = c * NSUB + s
    pltpu.sync_copy(col_hbm.at[wid], col_v)
    pltpu.sync_copy(z_hbm, hist)
    ones16 = jnp.full((16,), 1.0, jnp.float32)

    def body(i, carry):
      idx = col_v[pl.ds(i * 16, 16)]
      plsc.addupdate_scatter(hist, [idx], ones16)
      return carry

    lax.fori_loop(0, groups, body, 0)
    pltpu.sync_copy(hist, out_hbm.at[wid])

  return k(colf, zeros_row)


def _propagate(g, rowp4, colp4, zeros_slab, nacc, d, blocks, bchunks,
               bc0=None, bc1=None):
  rpt = nacc // NSUB
  nbuf = 2

  @functools.partial(
      pl.kernel, mesh=_sc_mesh(),
      out_type=jax.ShapeDtypeStruct((NCORES, nacc, d), jnp.float32),
      scratch_types=[
          pltpu.VMEM((bchunks, CHUNK), jnp.int32),
          pltpu.VMEM((bchunks, CHUNK), jnp.int32),
          pltpu.VMEM((nbuf * CHUNK, d), jnp.float32),
          pltpu.VMEM_SHARED((nacc, d), jnp.float32),
          pltpu.SemaphoreType.DMA((nbuf,)),
      ])
  def k(g_hbm, row_hbm, col_hbm, z_hbm, out_hbm, row_v, col_v, rows_v,
        acc, sems):
    c = lax.axis_index("c")
    s = lax.axis_index("s")
    wid = c * NSUB + s
    bcend = bchunks if bc0 is None else jnp.where(c == 0, bc0, bc1)
    pltpu.sync_copy(z_hbm, acc.at[pl.ds(s * rpt, rpt)])
    plsc.subcore_barrier()

    def buf(p):
      return rows_v.at[pl.ds(p * CHUNK, CHUNK)]

    def start_gather(j):
      p = lax.rem(j, nbuf)
      pltpu.async_copy(g_hbm.at[row_v.at[j]], buf(p), sems.at[p])

    for b in range(blocks):
      pltpu.sync_copy(row_hbm.at[wid, b], row_v)
      pltpu.sync_copy(col_hbm.at[wid, b], col_v)
      for jj in range(nbuf - 1):
        start_gather(jj)

      def body(j, carry):
        p = lax.rem(j, nbuf)

        @pl.when(j + nbuf - 1 < bcend)
        def _():
          start_gather(j + nbuf - 1)

        pltpu.make_async_copy(g_hbm.at[row_v.at[j]], buf(p),
                              sems.at[p]).wait()
        pltpu.sync_copy(buf(p), acc.at[col_v.at[j]], add=True)
        return carry

      lax.fori_loop(0, bcend, body, 0)

    plsc.subcore_barrier()
    pltpu.sync_copy(acc.at[pl.ds(s * rpt, rpt)],
                    out_hbm.at[c, pl.ds(s * rpt, rpt)])

  return k(g, rowp4, colp4, zeros_slab)


def _dinv_of(degp_ref, n):
  deg = jnp.sum(degp_ref[...], axis=1, keepdims=True) + 1.0
  return lax.rsqrt(deg)[:n]


def _tc_first(degp, x, w1):
  n, _ = x.shape
  h = w1.shape[1]

  def body(degp_ref, x_ref, w1_ref, g_ref):
    dinv = _dinv_of(degp_ref, n)
    hm = jnp.dot(x_ref[...], w1_ref[...], precision=lax.Precision.HIGHEST,
                 preferred_element_type=jnp.float32)
    g_ref[...] = dinv * hm

  return pl.pallas_call(
      body, out_shape=jax.ShapeDtypeStruct((n, h), jnp.float32))(degp, x, w1)


def _tc_mid(degp, s1, g1, b1r):
  n, h = g1.shape

  def body(degp_ref, s1_ref, g1_ref, b1_ref, u_ref):
    dinv = _dinv_of(degp_ref, n)
    s = s1_ref[0, :n, :] + s1_ref[1, :n, :]
    z = jnp.maximum(dinv * (s + g1_ref[...]) + b1_ref[...], 0.0)
    u_ref[...] = dinv * z

  return pl.pallas_call(
      body, out_shape=jax.ShapeDtypeStruct((n, h), jnp.float32))(
          degp, s1, g1, b1r)


def _tc_last(degp, s2, u, w2, b2r):
  n = u.shape[0]
  cdim = w2.shape[1]

  def body(degp_ref, s2_ref, u_ref, w2_ref, b2_ref, out_ref):
    dinv = _dinv_of(degp_ref, n)
    s = s2_ref[0, :n, :] + s2_ref[1, :n, :]
    p = dinv * (s + u_ref[...])
    out_ref[...] = jnp.dot(p, w2_ref[...], precision=lax.Precision.HIGHEST,
                           preferred_element_type=jnp.float32) + b2_ref[...]

  return pl.pallas_call(
      body, out_shape=jax.ShapeDtypeStruct((n, cdim), jnp.float32))(
          degp, s2, u, w2, b2r)


def kernel(x, edge_index, W1, b1, W2, b2):
  n, d = x.shape
  e = edge_index.shape[1]
  h = W1.shape[1]
  cdim = W2.shape[1]
  nacc = ((n + 1 + 8 * NSUB - 1) // (8 * NSUB)) * (8 * NSUB)
  blocks = 2
  rpt = nacc // NSUB

  bc0, bc1 = 52, 106
  e0 = NSUB * blocks * bc0 * CHUNK
  e1cap = NSUB * blocks * bc1 * CHUNK
  pad = e0 + e1cap - e

  row = edge_index[0].astype(jnp.int32)
  col = edge_index[1].astype(jnp.int32)
  rowp = jnp.concatenate([row, jnp.zeros((pad,), jnp.int32)])
  colp = jnp.concatenate([col, jnp.full((pad,), n, jnp.int32)])

  def _core_pack(arr, fill):
    a0 = arr[:e0].reshape(NSUB, blocks, bc0, CHUNK)
    a0 = jnp.pad(a0, ((0, 0), (0, 0), (0, bc1 - bc0), (0, 0)),
                 constant_values=fill)
    a1 = arr[e0:].reshape(NSUB, blocks, bc1, CHUNK)
    return jnp.concatenate([a0, a1], axis=0)

  rowp4 = _core_pack(rowp, 0)
  colp4 = _core_pack(colp, n)
  bchunks = bc1
  chunks = blocks * bchunks
  colf = colp4.reshape(NW, chunks * CHUNK)

  z1 = jnp.zeros((nacc,), jnp.float32)
  zd = jnp.zeros((rpt, h), jnp.float32)
  b1r = b1.reshape(1, h)
  b2r = b2.reshape(1, cdim)

  degp = _deg_partials(colf, z1, nacc, chunks)
  degp = jnp.swapaxes(degp, 0, 1)
  g1 = _tc_first(degp, x, W1)
  s1 = _propagate(g1, rowp4, colp4, zd, nacc, h, blocks, bchunks, bc0, bc1)
  u = _tc_mid(degp, s1, g1, b1r)
  s2 = _propagate(u, rowp4, colp4, zd, nacc, h, blocks, bchunks, bc0, bc1)
  return _tc_last(degp, s2, u, W2, b2r)


# speedup vs baseline: 1.1848x; 1.1848x over previous
import functools

import jax
import jax.numpy as jnp
from jax import lax
from jax.experimental import pallas as pl
from jax.experimental.pallas import tpu as pltpu
from jax.experimental.pallas import tpu_sc as plsc

NCORES = 2
NSUB = 16
NW = NCORES * NSUB
CHUNK = 64


def _sc_mesh():
  return plsc.VectorSubcoreMesh(
      core_axis_name="c", subcore_axis_name="s",
      num_cores=NCORES, num_subcores=NSUB)


def _deg_partials(colf, zeros_row, nacc, chunks):
  epw = chunks * CHUNK
  groups = epw // 16

  @functools.partial(
      pl.kernel, mesh=_sc_mesh(),
      out_type=jax.ShapeDtypeStruct((NW, nacc), jnp.float32),
      compiler_params=pltpu.CompilerParams(needs_layout_passes=False),
      scratch_types=[
          pltpu.VMEM((epw,), jnp.int32),
          pltpu.VMEM((nacc,), jnp.float32),
      ])
  def k(col_hbm, z_hbm, out_hbm, col_v, hist):
    c = lax.axis_index("c")
    s = lax.axis_index("s")
    wid = c * NSUB + s
    pltpu.sync_copy(col_hbm.at[wid], col_v)
    pltpu.sync_copy(z_hbm, hist)
    ones16 = jnp.full((16,), 1.0, jnp.float32)

    def body(i, carry):
      idx = col_v[pl.ds(i * 16, 16)]
      plsc.addupdate_scatter(hist, [idx], ones16)
      return carry

    lax.fori_loop(0, groups, body, 0)
    pltpu.sync_copy(hist, out_hbm.at[wid])

  return k(colf, zeros_row)


def _propagate(g, rowp4, colp4, zeros_slab, nacc, d, blocks, bchunks,
               bc0=None, bc1=None):
  rpt = nacc // NSUB
  nbuf = 2

  @functools.partial(
      pl.kernel, mesh=_sc_mesh(),
      out_type=jax.ShapeDtypeStruct((NCORES, nacc, d), jnp.float32),
      scratch_types=[
          pltpu.VMEM((bchunks, CHUNK), jnp.int32),
          pltpu.VMEM((bchunks, CHUNK), jnp.int32),
          pltpu.VMEM((nbuf * CHUNK, d), jnp.float32),
          pltpu.VMEM_SHARED((nacc, d), jnp.float32),
          pltpu.SemaphoreType.DMA((nbuf,)),
      ])
  def k(g_hbm, row_hbm, col_hbm, z_hbm, out_hbm, row_v, col_v, rows_v,
        acc, sems):
    c = lax.axis_index("c")
    s = lax.axis_index("s")
    wid = c * NSUB + s
    bcend = bchunks if bc0 is None else jnp.where(c == 0, bc0, bc1)
    pltpu.sync_copy(z_hbm, acc.at[pl.ds(s * rpt, rpt)])
    plsc.subcore_barrier()

    def buf(p):
      return rows_v.at[pl.ds(p * CHUNK, CHUNK)]

    def start_gather(j):
      p = lax.rem(j, nbuf)
      pltpu.async_copy(g_hbm.at[row_v.at[j]], buf(p), sems.at[p])

    for b in range(blocks):
      pltpu.sync_copy(row_hbm.at[wid, b], row_v)
      pltpu.sync_copy(col_hbm.at[wid, b], col_v)
      for jj in range(nbuf - 1):
        start_gather(jj)

      def body(j, carry):
        p = lax.rem(j, nbuf)

        @pl.when(j + nbuf - 1 < bcend)
        def _():
          start_gather(j + nbuf - 1)

        pltpu.make_async_copy(g_hbm.at[row_v.at[j]], buf(p),
                              sems.at[p]).wait()
        pltpu.sync_copy(buf(p), acc.at[col_v.at[j]], add=True)
        return carry

      lax.fori_loop(0, bcend, body, 0)

    plsc.subcore_barrier()
    pltpu.sync_copy(acc.at[pl.ds(s * rpt, rpt)],
                    out_hbm.at[c, pl.ds(s * rpt, rpt)])

  return k(g, rowp4, colp4, zeros_slab)


def _dinv_of(degp_ref, n):
  deg = jnp.sum(degp_ref[...], axis=1, keepdims=True) + 1.0
  return lax.rsqrt(deg)[:n]


def _tc_first(degp, x, w1):
  n, _ = x.shape
  h = w1.shape[1]

  def body(degp_ref, x_ref, w1_ref, g_ref):
    dinv = _dinv_of(degp_ref, n)
    hm = jnp.dot(x_ref[...], w1_ref[...], precision=lax.Precision.HIGHEST,
                 preferred_element_type=jnp.float32)
    g_ref[...] = dinv * hm

  return pl.pallas_call(
      body, out_shape=jax.ShapeDtypeStruct((n, h), jnp.float32))(degp, x, w1)


def _tc_mid(degp, s1, g1, b1r):
  n, h = g1.shape

  def body(degp_ref, s1_ref, g1_ref, b1_ref, u_ref):
    dinv = _dinv_of(degp_ref, n)
    s = s1_ref[0, :n, :] + s1_ref[1, :n, :]
    z = jnp.maximum(dinv * (s + g1_ref[...]) + b1_ref[...], 0.0)
    u_ref[...] = dinv * z

  return pl.pallas_call(
      body, out_shape=jax.ShapeDtypeStruct((n, h), jnp.float32))(
          degp, s1, g1, b1r)


def _tc_last(degp, s2, u, w2, b2r):
  n = u.shape[0]
  cdim = w2.shape[1]

  def body(degp_ref, s2_ref, u_ref, w2_ref, b2_ref, out_ref):
    dinv = _dinv_of(degp_ref, n)
    s = s2_ref[0, :n, :] + s2_ref[1, :n, :]
    p = dinv * (s + u_ref[...])
    out_ref[...] = jnp.dot(p, w2_ref[...], precision=lax.Precision.HIGHEST,
                           preferred_element_type=jnp.float32) + b2_ref[...]

  return pl.pallas_call(
      body, out_shape=jax.ShapeDtypeStruct((n, cdim), jnp.float32))(
          degp, s2, u, w2, b2r)


def kernel(x, edge_index, W1, b1, W2, b2):
  n, d = x.shape
  e = edge_index.shape[1]
  h = W1.shape[1]
  cdim = W2.shape[1]
  nacc = ((n + 1 + 8 * NSUB - 1) // (8 * NSUB)) * (8 * NSUB)
  blocks = 2
  rpt = nacc // NSUB

  bc0, bc1 = 106, 52
  e0 = NSUB * blocks * bc0 * CHUNK
  e1cap = NSUB * blocks * bc1 * CHUNK
  pad = e0 + e1cap - e

  row = edge_index[0].astype(jnp.int32)
  col = edge_index[1].astype(jnp.int32)
  rowp = jnp.concatenate([row, jnp.zeros((pad,), jnp.int32)])
  colp = jnp.concatenate([col, jnp.full((pad,), n, jnp.int32)])

  bcmax = max(bc0, bc1)

  def _core_pack(arr, fill):
    a0 = arr[:e0].reshape(NSUB, blocks, bc0, CHUNK)
    a0 = jnp.pad(a0, ((0, 0), (0, 0), (0, bcmax - bc0), (0, 0)),
                 constant_values=fill)
    a1 = arr[e0:].reshape(NSUB, blocks, bc1, CHUNK)
    a1 = jnp.pad(a1, ((0, 0), (0, 0), (0, bcmax - bc1), (0, 0)),
                 constant_values=fill)
    return jnp.concatenate([a0, a1], axis=0)

  rowp4 = _core_pack(rowp, 0)
  colp4 = _core_pack(colp, n)
  bchunks = bcmax
  chunks = blocks * bchunks
  colf = colp4.reshape(NW, chunks * CHUNK)

  z1 = jnp.zeros((nacc,), jnp.float32)
  zd = jnp.zeros((rpt, h), jnp.float32)
  b1r = b1.reshape(1, h)
  b2r = b2.reshape(1, cdim)

  degp = _deg_partials(colf, z1, nacc, chunks)
  degp = jnp.swapaxes(degp, 0, 1)
  g1 = _tc_first(degp, x, W1)
  s1 = _propagate(g1, rowp4, colp4, zd, nacc, h, blocks, bchunks, bc0, bc1)
  u = _tc_mid(degp, s1, g1, b1r)
  s2 = _propagate(u, rowp4, colp4, zd, nacc, h, blocks, bchunks, bc0, bc1)
  return _tc_last(degp, s2, u, W2, b2r)
